# baseline (device time: 421807 ns/iter reference)
import jax
import jax.numpy as jnp
from jax import lax
from jax.experimental import pallas as pl
from jax.experimental.pallas import tpu as pltpu

N_DEV = 32
M_CH = 128


def kernel(x, w_mat):
    m_total, k_shard = x.shape
    _, n = w_mat.shape

    def body(x_ref, w_ref, out_ref, comm_ref, send_sems, recv_sems):
        my = lax.axis_index("i")
        left = lax.rem(my - 1 + N_DEV, N_DEV)
        right = lax.rem(my + 1, N_DEV)

        barrier_sem = pltpu.get_barrier_semaphore()
        pl.semaphore_signal(
            barrier_sem, inc=1, device_id=(left,),
            device_id_type=pl.DeviceIdType.MESH,
        )
        pl.semaphore_signal(
            barrier_sem, inc=1, device_id=(right,),
            device_id_type=pl.DeviceIdType.MESH,
        )
        pl.semaphore_wait(barrier_sem, 2)

        for s in range(N_DEV - 1):
            slot = s % 2
            c = lax.rem(my - 1 - s + 2 * N_DEV, N_DEV)
            part = jnp.dot(
                x_ref[pl.ds(c * M_CH, M_CH), :],
                w_ref[:, :],
                preferred_element_type=jnp.float32,
            )
            if s == 0:
                comm_ref[slot] = part
            else:
                comm_ref[slot] = comm_ref[slot] + part
            rdma = pltpu.make_async_remote_copy(
                src_ref=comm_ref.at[slot],
                dst_ref=comm_ref.at[1 - slot],
                send_sem=send_sems.at[slot],
                recv_sem=recv_sems.at[1 - slot],
                device_id=(right,),
                device_id_type=pl.DeviceIdType.MESH,
            )
            rdma.start()
            rdma.wait()

        part = jnp.dot(
            x_ref[pl.ds(my * M_CH, M_CH), :],
            w_ref[:, :],
            preferred_element_type=jnp.float32,
        )
        out_ref[:, :] = comm_ref[1] + part

    return pl.pallas_call(
        body,
        out_shape=jax.ShapeDtypeStruct((M_CH, n), jnp.float32),
        in_specs=[
            pl.BlockSpec(memory_space=pltpu.VMEM),
            pl.BlockSpec(memory_space=pltpu.VMEM),
        ],
        out_specs=pl.BlockSpec(memory_space=pltpu.VMEM),
        scratch_shapes=[
            pltpu.VMEM((2, M_CH, n), jnp.float32),
            pltpu.SemaphoreType.DMA((2,)),
            pltpu.SemaphoreType.DMA((2,)),
        ],
        compiler_params=pltpu.CompilerParams(collective_id=0),
    )(x, w_mat)


# device time: 249398 ns/iter; 1.6913x vs baseline; 1.6913x over previous
import numpy as np
import jax
import jax.numpy as jnp
from jax import lax
from jax.experimental import pallas as pl
from jax.experimental.pallas import tpu as pltpu

N_DEV = 32
M_CH = 128
N_HALF = 1024

_LOG_ORDER = []
for _z in range(4):
    for _yi, _y in enumerate(range(4)):
        _row = [(0, _y, _z), (1, _y, _z)]
        if _yi % 2:
            _row.reverse()
        _LOG_ORDER.extend(_row)

_P = [(0, 0), (1, 0), (2, 0), (3, 0), (3, 1), (3, 2), (3, 3), (2, 3),
      (2, 2), (2, 1), (1, 1), (1, 2), (1, 3), (0, 3), (0, 2), (0, 1)]
_HAM = [(0, y, z) for y, z in _P] + [(1, y, z) for y, z in reversed(_P)]

_POS_OF_LOG = [_HAM.index(c) for c in _LOG_ORDER]
_RING_LOG = [_LOG_ORDER.index(c) for c in _HAM]

_NXT = np.array([_RING_LOG[(_POS_OF_LOG[d] + 1) % N_DEV] for d in range(N_DEV)],
                np.int32)
_PRV = np.array([_RING_LOG[(_POS_OF_LOG[d] - 1) % N_DEV] for d in range(N_DEV)],
                np.int32)
_CFWD = np.array([[_RING_LOG[(_POS_OF_LOG[d] - 1 - s) % N_DEV]
                   for s in range(N_DEV - 1)] for d in range(N_DEV)], np.int32)
_CREV = np.array([[_RING_LOG[(_POS_OF_LOG[d] + 1 + s) % N_DEV]
                   for s in range(N_DEV - 1)] for d in range(N_DEV)], np.int32)


def kernel(x, w_mat):
    m_total, k_shard = x.shape
    _, n = w_mat.shape

    my = lax.axis_index("i")
    nxt = jnp.take(jnp.asarray(_NXT), my).reshape(1)
    prv = jnp.take(jnp.asarray(_PRV), my).reshape(1)
    cfwd = lax.dynamic_index_in_dim(jnp.asarray(_CFWD), my, 0, keepdims=False)
    crev = lax.dynamic_index_in_dim(jnp.asarray(_CREV), my, 0, keepdims=False)

    def body(nxt_ref, prv_ref, cfwd_ref, crev_ref, x_ref, w_ref, out_ref,
             comm_f, comm_r, send_f, recv_f, send_r, recv_r):
        my_id = lax.axis_index("i")
        nxt_id = nxt_ref[0]
        prv_id = prv_ref[0]

        barrier_sem = pltpu.get_barrier_semaphore()
        pl.semaphore_signal(
            barrier_sem, inc=1, device_id=(nxt_id,),
            device_id_type=pl.DeviceIdType.MESH,
        )
        pl.semaphore_signal(
            barrier_sem, inc=1, device_id=(prv_id,),
            device_id_type=pl.DeviceIdType.MESH,
        )
        pl.semaphore_wait(barrier_sem, 2)

        for s in range(N_DEV - 1):
            slot = s % 2
            cf = cfwd_ref[s]
            cr = crev_ref[s]
            part_f = jnp.dot(
                x_ref[pl.ds(cf * M_CH, M_CH), :],
                w_ref[:, :N_HALF],
                preferred_element_type=jnp.float32,
            )
            part_r = jnp.dot(
                x_ref[pl.ds(cr * M_CH, M_CH), :],
                w_ref[:, N_HALF:],
                preferred_element_type=jnp.float32,
            )
            if s == 0:
                comm_f[slot] = part_f
                comm_r[slot] = part_r
            else:
                comm_f[slot] = comm_f[slot] + part_f
                comm_r[slot] = comm_r[slot] + part_r
            rdma_f = pltpu.make_async_remote_copy(
                src_ref=comm_f.at[slot],
                dst_ref=comm_f.at[1 - slot],
                send_sem=send_f.at[slot],
                recv_sem=recv_f.at[1 - slot],
                device_id=(nxt_id,),
                device_id_type=pl.DeviceIdType.MESH,
            )
            rdma_r = pltpu.make_async_remote_copy(
                src_ref=comm_r.at[slot],
                dst_ref=comm_r.at[1 - slot],
                send_sem=send_r.at[slot],
                recv_sem=recv_r.at[1 - slot],
                device_id=(prv_id,),
                device_id_type=pl.DeviceIdType.MESH,
            )
            rdma_f.start()
            rdma_r.start()
            rdma_f.wait()
            rdma_r.wait()

        part = jnp.dot(
            x_ref[pl.ds(my_id * M_CH, M_CH), :],
            w_ref[:, :],
            preferred_element_type=jnp.float32,
        )
        out_ref[:, :N_HALF] = comm_f[1] + part[:, :N_HALF]
        out_ref[:, N_HALF:] = comm_r[1] + part[:, N_HALF:]

    return pl.pallas_call(
        body,
        out_shape=jax.ShapeDtypeStruct((M_CH, n), jnp.float32),
        in_specs=[
            pl.BlockSpec(memory_space=pltpu.SMEM),
            pl.BlockSpec(memory_space=pltpu.SMEM),
            pl.BlockSpec(memory_space=pltpu.SMEM),
            pl.BlockSpec(memory_space=pltpu.SMEM),
            pl.BlockSpec(memory_space=pltpu.VMEM),
            pl.BlockSpec(memory_space=pltpu.VMEM),
        ],
        out_specs=pl.BlockSpec(memory_space=pltpu.VMEM),
        scratch_shapes=[
            pltpu.VMEM((2, M_CH, N_HALF), jnp.float32),
            pltpu.VMEM((2, M_CH, N_HALF), jnp.float32),
            pltpu.SemaphoreType.DMA((2,)),
            pltpu.SemaphoreType.DMA((2,)),
            pltpu.SemaphoreType.DMA((2,)),
            pltpu.SemaphoreType.DMA((2,)),
        ],
        compiler_params=pltpu.CompilerParams(collective_id=0),
    )(nxt, prv, cfwd, crev, x, w_mat)
